# Initial kernel scaffold; baseline (speedup 1.0000x reference)
import jax
import jax.numpy as jnp
from jax.experimental import pallas as pl


def kernel(x, edge_index, table, Wq, Wk, Wv, ln1_s, ln1_b, Wff, bff, ln2_s, ln2_b):
    raise NotImplementedError("write your pallas kernel here")


# speedup vs baseline: 19.0476x; 19.0476x over previous
import functools

import jax
import jax.numpy as jnp
import numpy as np
from jax import lax
from jax.experimental import pallas as pl
from jax.experimental.pallas import tpu as pltpu
from jax.experimental.pallas import tpu_sc as plsc

_N, _E, _V, _D, _H, _DH, _L = 10000, 160000, 64, 256, 4, 64, 3
_NC, _NS = 2, 16
_NW = _NC * _NS
_CH = 128
_NCHUNK = _E // _CH
_NP = 10240
_RPT = _NP // _NS
_f32 = jnp.float32

_BN = 2000
_BE = 2000


def _ln_rows(x, s, b):
    mu = jnp.mean(x, axis=-1, keepdims=True)
    xc = x - mu
    var = jnp.mean(xc * xc, axis=-1, keepdims=True)
    return xc * lax.rsqrt(var + 1e-5) * s + b


def _emb_body(x_ref, t_ref, o_ref):
    oh = (x_ref[...] == lax.broadcasted_iota(jnp.int32, (1, _V), 1)).astype(_f32)
    o_ref[...] = jnp.dot(oh, t_ref[...], preferred_element_type=_f32)


def _embed(x2d, table):
    return pl.pallas_call(
        _emb_body,
        grid=(_N // _BN,),
        in_specs=[pl.BlockSpec((_BN, 1), lambda i: (i, 0)),
                  pl.BlockSpec((_V, _D), lambda i: (0, 0))],
        out_specs=pl.BlockSpec((_BN, _D), lambda i: (i, 0)),
        out_shape=jax.ShapeDtypeStruct((_N, _D), _f32),
    )(x2d, table)


def _qkv_body(e_ref, wq_ref, wkv_ref, q_ref, kv_ref):
    e = e_ref[...]
    q_ref[...] = jnp.dot(e, wq_ref[...], preferred_element_type=_f32)
    kv_ref[...] = jnp.dot(e, wkv_ref[...], preferred_element_type=_f32)


def _qkv(emb, wq, wkv):
    return pl.pallas_call(
        _qkv_body,
        grid=(_N // _BN,),
        in_specs=[pl.BlockSpec((_BN, _D), lambda i: (i, 0)),
                  pl.BlockSpec((_D, _D), lambda i: (0, 0)),
                  pl.BlockSpec((_D, 2 * _D), lambda i: (0, 0))],
        out_specs=[pl.BlockSpec((_BN, _D), lambda i: (i, 0)),
                   pl.BlockSpec((_BN, 2 * _D), lambda i: (i, 0))],
        out_shape=[jax.ShapeDtypeStruct((_N, _D), _f32),
                   jax.ShapeDtypeStruct((_N, 2 * _D), _f32)],
    )(emb, wq, wkv)


def _edge_body(qd_ref, kvs_ref, wv_ref, ex_ref):
    q = qd_ref[...]
    k = kvs_ref[:, :_D]
    v = kvs_ref[:, _D:]
    s = q * k
    scale = 1.0 / np.sqrt(_DH)
    wvs, exs = [], []
    for h in range(_H):
        lh = jnp.sum(s[:, h * _DH:(h + 1) * _DH], axis=1, keepdims=True) * scale
        eh = jnp.exp(lh)
        exs.append(eh)
        wvs.append(v[:, h * _DH:(h + 1) * _DH] * eh)
    wv_ref[...] = jnp.concatenate(wvs, axis=1)
    ex_ref[...] = jnp.concatenate(exs + [jnp.zeros((q.shape[0], 124), _f32)], axis=1)


def _edge(qd, kvs):
    return pl.pallas_call(
        _edge_body,
        grid=(_E // _BE,),
        in_specs=[pl.BlockSpec((_BE, _D), lambda i: (i, 0)),
                  pl.BlockSpec((_BE, 2 * _D), lambda i: (i, 0))],
        out_specs=[pl.BlockSpec((_BE, _D), lambda i: (i, 0)),
                   pl.BlockSpec((_BE, 128), lambda i: (i, 0))],
        out_shape=[jax.ShapeDtypeStruct((_E, _D), _f32),
                   jax.ShapeDtypeStruct((_E, 128), _f32)],
    )(qd, kvs)


def _post_body(e_ref, a_ref, d_ref, l1s_ref, l1b_ref, w0_ref, b0_ref,
               w1_ref, b1_ref, l2s_ref, l2b_ref, o_ref):
    e = e_ref[...]
    a = a_ref[...]
    d = d_ref[0] + d_ref[1]
    parts = [a[:, h * _DH:(h + 1) * _DH] / (d[:, h:h + 1] + 1e-16)
             for h in range(_H)]
    agg = jnp.concatenate(parts, axis=1)
    hh = _ln_rows(e + agg, l1s_ref[...], l1b_ref[...])
    ff = jnp.dot(hh, w0_ref[...], preferred_element_type=_f32) + b0_ref[...]
    ff = jnp.maximum(ff, 0.0)
    ff = jnp.dot(ff, w1_ref[...], preferred_element_type=_f32) + b1_ref[...]
    o_ref[...] = _ln_rows(hh + ff, l2s_ref[...], l2b_ref[...])


def _post(emb, agg, den, l1s, l1b, w0, b0, w1, b1, l2s, l2b):
    row = lambda i: (0, 0)
    return pl.pallas_call(
        _post_body,
        grid=(_N // _BN,),
        in_specs=[pl.BlockSpec((_BN, _D), lambda i: (i, 0)),
                  pl.BlockSpec((_BN, _D), lambda i: (i, 0)),
                  pl.BlockSpec((2, _BN, 128), lambda i: (0, i, 0)),
                  pl.BlockSpec((1, _D), row),
                  pl.BlockSpec((1, _D), row),
                  pl.BlockSpec((_D, _D), row),
                  pl.BlockSpec((1, _D), row),
                  pl.BlockSpec((_D, _D), row),
                  pl.BlockSpec((1, _D), row),
                  pl.BlockSpec((1, _D), row),
                  pl.BlockSpec((1, _D), row)],
        out_specs=pl.BlockSpec((_BN, _D), lambda i: (i, 0)),
        out_shape=jax.ShapeDtypeStruct((_N, _D), _f32),
    )(emb, agg, den, l1s, l1b, w0, b0, w1, b1, l2s, l2b)


def _sc_mesh():
    return plsc.VectorSubcoreMesh(core_axis_name="c", subcore_axis_name="s")


def _gather_body(q_hbm, kv_hbm, dst_hbm, src_hbm, qd_out, kvs_out,
                 idx_v, bufq, bufkv, sem):
    c = lax.axis_index("c")
    s = lax.axis_index("s")
    wid = s * _NC + c
    nch = jnp.where(wid < _NCHUNK % _NW, _NCHUNK // _NW + 1, _NCHUNK // _NW)

    def body(j, carry):
        off = (wid + j * _NW) * _CH
        pltpu.sync_copy(dst_hbm.at[pl.ds(off, _CH)], idx_v)
        pltpu.async_copy(q_hbm.at[idx_v], bufq, sem).wait()
        pltpu.sync_copy(bufq, qd_out.at[pl.ds(off, _CH)])
        pltpu.sync_copy(src_hbm.at[pl.ds(off, _CH)], idx_v)
        pltpu.async_copy(kv_hbm.at[idx_v], bufkv, sem).wait()
        pltpu.sync_copy(bufkv, kvs_out.at[pl.ds(off, _CH)])
        return carry

    lax.fori_loop(0, nch, body, 0)


def _gather(q, kv, dst, src):
    f = pl.kernel(
        _gather_body,
        out_type=(jax.ShapeDtypeStruct((_E, _D), _f32),
                  jax.ShapeDtypeStruct((_E, 2 * _D), _f32)),
        mesh=_sc_mesh(),
        scratch_types=[pltpu.VMEM((_CH,), jnp.int32),
                       pltpu.VMEM((_CH, _D), _f32),
                       pltpu.VMEM((_CH, 2 * _D), _f32),
                       pltpu.SemaphoreType.DMA],
    )
    return f(q, kv, dst, src)


def _scatter_agg_body(dst_hbm, wv_hbm, z_hbm, agg_out, agg_sp, idx_v, wbuf):
    c = lax.axis_index("c")
    s = lax.axis_index("s")
    pltpu.sync_copy(z_hbm, wbuf)

    def initb(t, carry):
        pltpu.sync_copy(wbuf, agg_sp.at[pl.ds(s * _RPT + t * _CH, _CH)])
        return carry

    lax.fori_loop(0, _RPT // _CH, initb, 0)
    plsc.subcore_barrier()
    nch = jnp.where(s < _NCHUNK % _NS, _NCHUNK // _NS + 1, _NCHUNK // _NS)

    def body(j, carry):
        off = (s + j * _NS) * _CH
        pltpu.sync_copy(dst_hbm.at[pl.ds(off, _CH)], idx_v)
        pltpu.sync_copy(wv_hbm.at[pl.ds(off, _CH), pl.ds(c * 128, 128)], wbuf)
        pltpu.sync_copy(wbuf, agg_sp.at[idx_v], add=True)
        return carry

    lax.fori_loop(0, nch, body, 0)
    plsc.subcore_barrier()

    def dumpb(t, carry):
        sub = pl.ds(s * _RPT + t * _CH, _CH)
        pltpu.sync_copy(agg_sp.at[sub], wbuf)
        pltpu.sync_copy(wbuf, agg_out.at[sub, pl.ds(c * 128, 128)])
        return carry

    lax.fori_loop(0, _RPT // _CH, dumpb, 0)


def _scatter_agg(dst, wv, z):
    f = pl.kernel(
        _scatter_agg_body,
        out_type=jax.ShapeDtypeStruct((_NP, _D), _f32),
        mesh=_sc_mesh(),
        scratch_types=[pltpu.VMEM_SHARED((_NP, 128), _f32),
                       pltpu.VMEM((_CH,), jnp.int32),
                       pltpu.VMEM((_CH, 128), _f32)],
    )
    return f(dst, wv, z)


def _scatter_den_body(dst_hbm, ex_hbm, z_hbm, den_out, den_sp, idx_v, ebuf):
    c = lax.axis_index("c")
    s = lax.axis_index("s")
    pltpu.sync_copy(z_hbm, ebuf)

    def initb(t, carry):
        pltpu.sync_copy(ebuf, den_sp.at[pl.ds(s * _RPT + t * _CH, _CH)])
        return carry

    lax.fori_loop(0, _RPT // _CH, initb, 0)
    plsc.subcore_barrier()
    half = _NCHUNK // _NC
    nch = jnp.where(s < half % _NS, half // _NS + 1, half // _NS)

    def body(j, carry):
        off = (c * half + s + j * _NS) * _CH
        pltpu.sync_copy(dst_hbm.at[pl.ds(off, _CH)], idx_v)
        pltpu.sync_copy(ex_hbm.at[pl.ds(off, _CH), pl.ds(0, 128)], ebuf)
        pltpu.sync_copy(ebuf, den_sp.at[idx_v], add=True)
        return carry

    lax.fori_loop(0, nch, body, 0)
    plsc.subcore_barrier()

    def dumpb(t, carry):
        sub = pl.ds(s * _RPT + t * _CH, _CH)
        pltpu.sync_copy(den_sp.at[sub], ebuf)
        pltpu.sync_copy(ebuf, den_out.at[c, sub])
        return carry

    lax.fori_loop(0, _RPT // _CH, dumpb, 0)


def _scatter_den(dst, ex, z):
    f = pl.kernel(
        _scatter_den_body,
        out_type=jax.ShapeDtypeStruct((_NC, _NP, 128), _f32),
        mesh=_sc_mesh(),
        scratch_types=[pltpu.VMEM_SHARED((_NP, 128), _f32),
                       pltpu.VMEM((_CH,), jnp.int32),
                       pltpu.VMEM((_CH, 128), _f32)],
    )
    return f(dst, ex, z)


def kernel(x, edge_index, table, Wq, Wk, Wv, ln1_s, ln1_b, Wff, bff, ln2_s, ln2_b):
    x2 = x.reshape(_N, 1).astype(jnp.int32)
    src = edge_index[0].astype(jnp.int32)
    dst = edge_index[1].astype(jnp.int32)
    z = jnp.zeros((_CH, 128), _f32)
    emb = _embed(x2, table)
    for i in range(_L):
        wkv = jnp.concatenate([Wk[i], Wv[i]], axis=1)
        q, kv = _qkv(emb, Wq[i], wkv)
        qd, kvs = _gather(q, kv, dst, src)
        wv, ex = _edge(qd, kvs)
        agg = _scatter_agg(dst, wv, z)
        den2 = _scatter_den(dst, ex, z)
        emb = _post(emb, agg, den2,
                    ln1_s[i].reshape(1, _D), ln1_b[i].reshape(1, _D),
                    Wff[i, 0], bff[i, 0].reshape(1, _D),
                    Wff[i, 1], bff[i, 1].reshape(1, _D),
                    ln2_s[i].reshape(1, _D), ln2_b[i].reshape(1, _D))
    return emb
